# Initial kernel scaffold; baseline (speedup 1.0000x reference)
import jax
import jax.numpy as jnp
from jax.experimental import pallas as pl


def kernel(flat_adj_matrix, flat_adj_matrix_t, W_gnn, b_gnn, W_mlp, b_mlp):
    raise NotImplementedError("write your pallas kernel here")


# speedup vs baseline: 1.9252x; 1.9252x over previous
import jax
import jax.numpy as jnp
from jax.experimental import pallas as pl
from jax.experimental.pallas import tpu as pltpu

N_NODES = 256
IN_DIM = N_NODES * N_NODES
HIDDEN = 256
N_CELLS = 3
BATCH = 4

K_BLK = 4096
KC = IN_DIM // K_BLK


def _body(x_ref, w_ref, bg_ref, wm_ref, bm_ref, o_ref, acc_ref):
    i = pl.program_id(0)
    k = pl.program_id(1)

    @pl.when(k == 0)
    def _init():
        acc_ref[i] = jnp.zeros((2 * BATCH, HIDDEN), jnp.float32)

    acc_ref[i] += jnp.dot(
        x_ref[...], w_ref[0], preferred_element_type=jnp.float32
    )

    @pl.when((i == N_CELLS - 1) & (k == KC - 1))
    def _epilogue():
        z0 = jnp.maximum(acc_ref[0] + bg_ref[0:1, :], 0.0)
        z1 = jnp.maximum(acc_ref[1] + bg_ref[1:2, :], 0.0) * z0
        z2 = jnp.maximum(acc_ref[2] + bg_ref[2:3, :], 0.0) * z1
        zs = z0 + z1 + z2
        s = jnp.dot(zs, wm_ref[...], preferred_element_type=jnp.float32)
        s = s + 3.0 * bm_ref[...]
        o_ref[...] = s[:BATCH] * s[BATCH:]


def kernel(flat_adj_matrix, flat_adj_matrix_t, W_gnn, b_gnn, W_mlp, b_mlp):
    x = jnp.concatenate([flat_adj_matrix, flat_adj_matrix_t], axis=0)
    bm = b_mlp.reshape(1, N_NODES)

    grid = (N_CELLS, KC)
    return pl.pallas_call(
        _body,
        grid=grid,
        in_specs=[
            pl.BlockSpec((2 * BATCH, K_BLK), lambda i, k: (0, k)),
            pl.BlockSpec((1, K_BLK, HIDDEN), lambda i, k: (i, k, 0)),
            pl.BlockSpec((N_CELLS, HIDDEN), lambda i, k: (0, 0)),
            pl.BlockSpec((HIDDEN, N_NODES), lambda i, k: (0, 0)),
            pl.BlockSpec((1, N_NODES), lambda i, k: (0, 0)),
        ],
        out_specs=pl.BlockSpec((BATCH, N_NODES), lambda i, k: (0, 0)),
        out_shape=jax.ShapeDtypeStruct((BATCH, N_NODES), jnp.float32),
        scratch_shapes=[pltpu.VMEM((N_CELLS, 2 * BATCH, HIDDEN), jnp.float32)],
        compiler_params=pltpu.CompilerParams(
            dimension_semantics=("arbitrary", "arbitrary"),
        ),
    )(x, W_gnn, b_gnn, W_mlp, bm)


# speedup vs baseline: 2.2887x; 1.1888x over previous
import jax
import jax.numpy as jnp
from jax.experimental import pallas as pl
from jax.experimental.pallas import tpu as pltpu

N_NODES = 256
IN_DIM = N_NODES * N_NODES
HIDDEN = 256
N_CELLS = 3
BATCH = 4

K_BLK = 8192
KC = IN_DIM // K_BLK


def _body(x_ref, xt_ref, w_ref, bg_ref, wm_ref, bm_ref, o_ref, acc_ref):
    i = pl.program_id(0)
    k = pl.program_id(1)

    @pl.when(k == 0)
    def _init():
        acc_ref[i] = jnp.zeros((2 * BATCH, HIDDEN), jnp.float32)

    xx = jnp.concatenate([x_ref[...], xt_ref[...]], axis=0)
    acc_ref[i] += jnp.dot(
        xx, w_ref[0], preferred_element_type=jnp.float32
    )

    @pl.when((i == N_CELLS - 1) & (k == KC - 1))
    def _epilogue():
        z0 = jnp.maximum(acc_ref[0] + bg_ref[0:1, :], 0.0)
        z1 = jnp.maximum(acc_ref[1] + bg_ref[1:2, :], 0.0) * z0
        z2 = jnp.maximum(acc_ref[2] + bg_ref[2:3, :], 0.0) * z1
        zs = z0 + z1 + z2
        s = jnp.dot(zs, wm_ref[...], preferred_element_type=jnp.float32)
        s = s + 3.0 * bm_ref[...]
        o_ref[...] = s[:BATCH] * s[BATCH:]


def kernel(flat_adj_matrix, flat_adj_matrix_t, W_gnn, b_gnn, W_mlp, b_mlp):
    bm = b_mlp.reshape(1, N_NODES)

    grid = (N_CELLS, KC)
    return pl.pallas_call(
        _body,
        grid=grid,
        in_specs=[
            pl.BlockSpec((BATCH, K_BLK), lambda i, k: (0, k)),
            pl.BlockSpec((BATCH, K_BLK), lambda i, k: (0, k)),
            pl.BlockSpec((1, K_BLK, HIDDEN), lambda i, k: (i, k, 0)),
            pl.BlockSpec((N_CELLS, HIDDEN), lambda i, k: (0, 0)),
            pl.BlockSpec((HIDDEN, N_NODES), lambda i, k: (0, 0)),
            pl.BlockSpec((1, N_NODES), lambda i, k: (0, 0)),
        ],
        out_specs=pl.BlockSpec((BATCH, N_NODES), lambda i, k: (0, 0)),
        out_shape=jax.ShapeDtypeStruct((BATCH, N_NODES), jnp.float32),
        scratch_shapes=[pltpu.VMEM((N_CELLS, 2 * BATCH, HIDDEN), jnp.float32)],
        compiler_params=pltpu.CompilerParams(
            dimension_semantics=("arbitrary", "arbitrary"),
        ),
    )(flat_adj_matrix, flat_adj_matrix_t, W_gnn, b_gnn, W_mlp, bm)
